# baseline (speedup 1.0000x reference)
import functools

import jax
import jax.numpy as jnp
from jax import lax
from jax.experimental import pallas as pl
from jax.experimental.pallas import tpu as pltpu
from jax.experimental.pallas import tpu_sc as plsc

CHUNK = 32
UNROLL = 8


def kernel(x, pe):
    batch, seq_len, dim = x.shape
    info = plsc.get_sparse_core_info()
    n_workers = info.num_cores * info.num_subcores
    s_per_w = seq_len // n_workers
    nchunks = s_per_w // CHUNK
    mesh = plsc.VectorSubcoreMesh(core_axis_name="c", subcore_axis_name="s")

    @functools.partial(
        pl.kernel,
        mesh=mesh,
        out_type=jax.ShapeDtypeStruct((batch * seq_len, dim), x.dtype),
        scratch_types=[
            pltpu.VMEM((CHUNK, dim), jnp.float32),
            pltpu.VMEM((CHUNK, dim), jnp.float32),
            pltpu.VMEM((CHUNK, dim), jnp.float32),
            pltpu.SemaphoreType.DMA,
            pltpu.SemaphoreType.DMA,
        ],
    )
    def sc_add(x_hbm, pe_hbm, out_hbm, xb0, xb1, pbuf, sx0, sx1):
        wid = lax.axis_index("s") * info.num_cores + lax.axis_index("c")
        rbase = wid * s_per_w

        xb = (xb0, xb1)
        sx = (sx0, sx1)

        def row_off(c, b):
            return b * seq_len + rbase + c * CHUNK

        def issue_in(p, c, b):
            pltpu.async_copy(
                x_hbm.at[pl.ds(row_off(c, b), CHUNK), :], xb[p], sx[p])

        def wait_in(p):
            pltpu.make_async_copy(
                x_hbm.at[pl.ds(0, CHUNK), :], xb[p], sx[p]).wait()

        def add_chunk(p):
            def row(r, carry):
                def col(jv, carry2):
                    sl = pl.ds(jv * 16, 16)
                    xb[p][r, sl] = xb[p][r, sl] + pbuf[r, sl]
                    return carry2
                lax.fori_loop(0, dim // 16, col, 0, unroll=UNROLL)
                return carry
            lax.fori_loop(0, CHUNK, row, 0)

        issue_in(0, 0, 0)
        issue_in(1, 0, 1)

        def chunk_body(c, carry):
            pltpu.sync_copy(pe_hbm.at[pl.ds(rbase + c * CHUNK, CHUNK), :],
                            pbuf)
            for b in range(batch):
                p = b % 2
                wait_in(p)
                add_chunk(p)
                pltpu.sync_copy(xb[p], out_hbm.at[pl.ds(row_off(c, b),
                                                        CHUNK), :])
                if b < 2:
                    issue_in(p, c, b + 2)
                else:
                    @pl.when(c < nchunks - 1)
                    def _():
                        issue_in(p, c + 1, b - 2)
            return carry

        lax.fori_loop(0, nchunks, chunk_body, 0)

    out = sc_add(x.reshape(batch * seq_len, dim), pe[:seq_len])
    return out.reshape(batch, seq_len, dim)


# speedup vs baseline: 4.2797x; 4.2797x over previous
import jax
import jax.numpy as jnp
from jax.experimental import pallas as pl
from jax.experimental.pallas import tpu as pltpu

BLOCK_S = 512


def _add_kernel(x_ref, pe_ref, out_ref):
    out_ref[...] = x_ref[...] + pe_ref[...][None, :, :]


def kernel(x, pe):
    batch, seq_len, dim = x.shape
    grid = (seq_len // BLOCK_S,)
    return pl.pallas_call(
        _add_kernel,
        grid=grid,
        in_specs=[
            pl.BlockSpec((batch, BLOCK_S, dim), lambda i: (0, i, 0)),
            pl.BlockSpec((BLOCK_S, dim), lambda i: (i, 0)),
        ],
        out_specs=pl.BlockSpec((batch, BLOCK_S, dim), lambda i: (0, i, 0)),
        out_shape=jax.ShapeDtypeStruct((batch, seq_len, dim), x.dtype),
        compiler_params=pltpu.CompilerParams(
            vmem_limit_bytes=100 * 1024 * 1024,
        ),
    )(x, pe[:seq_len])
